# Initial kernel scaffold; baseline (speedup 1.0000x reference)
import functools

import jax
import jax.numpy as jnp
from jax import lax
from jax.experimental import pallas as pl
from jax.experimental.pallas import tpu as pltpu
from jax.experimental.pallas import tpu_sc as plsc

EMB_DIM = 64
NEG_K = 20
NC, NS, LANES = 2, 16, 16
NW = NC * NS
CHUNK = 32
NEG_ROWS = CHUNK * NEG_K
NEG_IDX_ROWS = NEG_ROWS // 128


def _sc_scores(pos_u, pos_v, neg2d, u_weight, v_weight):
    B = pos_u.shape[0]
    per_w = B // NW
    n_chunks = per_w // CHUNK
    mesh = plsc.VectorSubcoreMesh(core_axis_name="c", subcore_axis_name="s")

    @functools.partial(
        pl.kernel,
        out_type=[
            jax.ShapeDtypeStruct((B,), jnp.float32),
            jax.ShapeDtypeStruct((B, NEG_K), jnp.float32),
        ],
        mesh=mesh,
        scratch_types=[
            pltpu.VMEM((CHUNK,), jnp.int32),
            pltpu.VMEM((CHUNK,), jnp.int32),
            pltpu.VMEM((NEG_IDX_ROWS, 128), jnp.int32),
            pltpu.VMEM((CHUNK, EMB_DIM), jnp.float32),
            pltpu.VMEM((CHUNK, EMB_DIM), jnp.float32),
            pltpu.VMEM((NEG_ROWS, EMB_DIM), jnp.float32),
            pltpu.VMEM((CHUNK,), jnp.float32),
            pltpu.VMEM((CHUNK, NEG_K), jnp.float32),
            pltpu.SemaphoreType.DMA,
        ],
    )
    def sc_kernel(pos_u_hbm, pos_v_hbm, neg_hbm, uw_hbm, vw_hbm,
                  pos_out, neg_out,
                  idxu, idxv, idxn, urows, vrows, nrows, psco, nsco, sem):
        wid = lax.axis_index("s") * NC + lax.axis_index("c")

        @pl.loop(0, n_chunks)
        def _chunk(ci):
            base = wid * per_w + ci * CHUNK
            pltpu.sync_copy(pos_u_hbm.at[pl.ds(base, CHUNK)], idxu)
            pltpu.sync_copy(pos_v_hbm.at[pl.ds(base, CHUNK)], idxv)
            r0 = base * NEG_K // 128
            pltpu.sync_copy(neg_hbm.at[pl.ds(r0, NEG_IDX_ROWS)], idxn)

            copies = [
                pltpu.async_copy(uw_hbm.at[idxu], urows, sem),
                pltpu.async_copy(vw_hbm.at[idxv], vrows, sem),
            ]
            for j in range(NEG_IDX_ROWS):
                copies.append(pltpu.async_copy(
                    vw_hbm.at[idxn.at[j]],
                    nrows.at[pl.ds(j * 128, 128)], sem))
            for c in copies:
                c.wait()

            @pl.loop(0, CHUNK)
            def _elem(e):
                u = [urows[e, pl.ds(16 * j, 16)] for j in range(4)]
                v = [vrows[e, pl.ds(16 * j, 16)] for j in range(4)]
                acc = u[0] * v[0] + u[1] * v[1] + u[2] * v[2] + u[3] * v[3]
                psco[e] = jnp.sum(acc)
                for kk in range(NEG_K):
                    row = e * NEG_K + kk
                    n = [nrows[row, pl.ds(16 * j, 16)] for j in range(4)]
                    a = u[0] * n[0] + u[1] * n[1] + u[2] * n[2] + u[3] * n[3]
                    nsco[e, kk] = jnp.sum(a)

            pltpu.sync_copy(psco, pos_out.at[pl.ds(base, CHUNK)])
            pltpu.sync_copy(nsco, neg_out.at[pl.ds(base, CHUNK)])

    return sc_kernel(pos_u, pos_v, neg2d, u_weight, v_weight)


def _tc_loss(pos_s, neg_s, batch):
    def body(p_ref, n_ref, o_ref):
        s = jnp.clip(p_ref[...], -10.0, 10.0)
        t1 = jnp.sum(-jax.nn.log_sigmoid(s))
        ns = jnp.clip(n_ref[...], -10.0, 10.0)
        t2 = jnp.sum(-jax.nn.log_sigmoid(-ns))
        o_ref[0, 0] = (t1 + t2) / batch

    return pl.pallas_call(
        body,
        out_shape=jax.ShapeDtypeStruct((1, 1), jnp.float32),
    )(pos_s, neg_s)


def kernel(pos_u, pos_v, neg_v, u_weight, v_weight):
    B = pos_u.shape[0]
    pos_u = pos_u.astype(jnp.int32)
    pos_v = pos_v.astype(jnp.int32)
    neg2d = neg_v.astype(jnp.int32).reshape(-1, 128)
    pos_s, neg_s = _sc_scores(pos_u, pos_v, neg2d, u_weight, v_weight)
    loss = _tc_loss(pos_s.reshape(128, -1), neg_s.reshape(-1, 128), float(B))
    return loss[0, 0]


# speedup vs baseline: 4.0258x; 4.0258x over previous
import dataclasses
import functools

import jax
import jax.numpy as jnp
from jax import lax
from jax.experimental import pallas as pl
from jax.experimental.pallas import tpu as pltpu
from jax.experimental.pallas import tpu_sc as plsc

EMB_DIM = 64
NEG_K = 20
NC, NS, LANES = 2, 16, 16
NW = NC * NS
CHUNK = 32
NEG_ROWS = CHUNK * NEG_K
NEG_IDX_ROWS = NEG_ROWS // 128
KQ = 4


def _sc_scores(pos_u, pos_v, neg2d, u_weight, v_weight):
    B = pos_u.shape[0]
    per_w = B // NW
    n_chunks = per_w // CHUNK
    mesh = plsc.VectorSubcoreMesh(core_axis_name="c", subcore_axis_name="s")
    cp = pltpu.CompilerParams()
    if "needs_layout_passes" in pltpu.CompilerParams.__dataclass_fields__:
        cp = dataclasses.replace(cp, needs_layout_passes=False)
    if "use_tc_tiling_on_sc" in pltpu.CompilerParams.__dataclass_fields__:
        cp = dataclasses.replace(cp, use_tc_tiling_on_sc=False)

    @functools.partial(
        pl.kernel,
        compiler_params=cp,
        out_type=[
            jax.ShapeDtypeStruct((B,), jnp.float32),
            jax.ShapeDtypeStruct((B * NEG_K,), jnp.float32),
        ],
        mesh=mesh,
        scratch_types=[
            pltpu.VMEM((CHUNK,), jnp.int32),
            pltpu.VMEM((CHUNK,), jnp.int32),
            pltpu.VMEM((NEG_ROWS,), jnp.int32),
            pltpu.VMEM((CHUNK, EMB_DIM), jnp.float32),
            pltpu.VMEM((CHUNK, EMB_DIM), jnp.float32),
            pltpu.VMEM((NEG_ROWS, EMB_DIM), jnp.float32),
            pltpu.VMEM((CHUNK,), jnp.float32),
            pltpu.VMEM((NEG_ROWS,), jnp.float32),
            pltpu.SemaphoreType.DMA,
        ],
    )
    def sc_kernel(pos_u_hbm, pos_v_hbm, neg_hbm, uw_hbm, vw_hbm,
                  pos_out, neg_out,
                  idxu, idxv, idxn, urows, vrows, nrows, psco, nsco, sem):
        wid = lax.axis_index("s") * NC + lax.axis_index("c")
        lane = lax.iota(jnp.int32, LANES)
        zero_col = lane * 0

        @pl.loop(0, n_chunks)
        def _chunk(ci):
            base = wid * per_w + ci * CHUNK
            pltpu.sync_copy(pos_u_hbm.at[pl.ds(base, CHUNK)], idxu)
            pltpu.sync_copy(pos_v_hbm.at[pl.ds(base, CHUNK)], idxv)
            pltpu.sync_copy(neg_hbm.at[pl.ds(base * NEG_K, NEG_ROWS)], idxn)

            copies = [
                pltpu.async_copy(uw_hbm.at[idxu], urows, sem),
                pltpu.async_copy(vw_hbm.at[idxv], vrows, sem),
            ]
            for j in range(NEG_IDX_ROWS):
                copies.append(pltpu.async_copy(
                    vw_hbm.at[idxn.at[pl.ds(j * 128, 128)]],
                    nrows.at[pl.ds(j * 128, 128)], sem))
            for c in copies:
                c.wait()

            for g in range(CHUNK // LANES):
                urow = lane + (g * LANES)

                accp = jnp.zeros((LANES,), jnp.float32)
                col = zero_col
                for _f in range(EMB_DIM):
                    gu = plsc.load_gather(urows, [urow, col])
                    gv = plsc.load_gather(vrows, [urow, col])
                    accp = accp + gu * gv
                    col = col + 1
                psco[pl.ds(g * LANES, LANES)] = accp

                sbase = lane * NEG_K + (g * LANES * NEG_K)

                @pl.loop(0, NEG_K, step=KQ)
                def _negs(k):
                    accs = [jnp.zeros((LANES,), jnp.float32) for _ in range(KQ)]
                    nrow = [urow * NEG_K + (k + q) for q in range(KQ)]
                    col = zero_col
                    for _f in range(EMB_DIM):
                        gu = plsc.load_gather(urows, [urow, col])
                        for q in range(KQ):
                            gn = plsc.load_gather(nrows, [nrow[q], col])
                            accs[q] = accs[q] + gu * gn
                        col = col + 1
                    for q in range(KQ):
                        plsc.store_scatter(nsco, [sbase + (k + q)], accs[q])

            pltpu.sync_copy(psco, pos_out.at[pl.ds(base, CHUNK)])
            pltpu.sync_copy(nsco, neg_out.at[pl.ds(base * NEG_K, NEG_ROWS)])

    return sc_kernel(pos_u, pos_v, neg2d, u_weight, v_weight)


def _tc_loss(pos_s, neg_s, batch):
    def body(p_ref, n_ref, o_ref):
        s = jnp.clip(p_ref[...], -10.0, 10.0)
        t1 = jnp.sum(-jax.nn.log_sigmoid(s))
        ns = jnp.clip(n_ref[...], -10.0, 10.0)
        t2 = jnp.sum(-jax.nn.log_sigmoid(-ns))
        o_ref[...] = jnp.reshape((t1 + t2) / batch, (1, 1))

    return pl.pallas_call(
        body,
        out_shape=jax.ShapeDtypeStruct((1, 1), jnp.float32),
    )(pos_s, neg_s)


def kernel(pos_u, pos_v, neg_v, u_weight, v_weight):
    B = pos_u.shape[0]
    pos_u = pos_u.astype(jnp.int32)
    pos_v = pos_v.astype(jnp.int32)
    neg_flat = neg_v.astype(jnp.int32).reshape(-1)
    pos_s, neg_s = _sc_scores(pos_u, pos_v, neg_flat, u_weight, v_weight)
    loss = _tc_loss(pos_s.reshape(128, -1), neg_s.reshape(-1, 128), float(B))
    return loss[0, 0]


# speedup vs baseline: 5.3622x; 1.3320x over previous
import dataclasses
import functools

import jax
import jax.numpy as jnp
from jax import lax
from jax.experimental import pallas as pl
from jax.experimental.pallas import tpu as pltpu
from jax.experimental.pallas import tpu_sc as plsc

EMB_DIM = 64
NEG_K = 20
NC, NS, LANES = 2, 16, 16
NW = NC * NS
CHUNK = 32
NEG_ROWS = CHUNK * NEG_K
GATHER_W = 128
KQ = 4


def _sc_scores(pos_u, pos_v, neg_flat, u_weight, v_weight):
    B = pos_u.shape[0]
    per_w = B // NW
    n_chunks = per_w // CHUNK
    mesh = plsc.VectorSubcoreMesh(core_axis_name="c", subcore_axis_name="s")
    cp = pltpu.CompilerParams()
    if "needs_layout_passes" in pltpu.CompilerParams.__dataclass_fields__:
        cp = dataclasses.replace(cp, needs_layout_passes=False)
    if "use_tc_tiling_on_sc" in pltpu.CompilerParams.__dataclass_fields__:
        cp = dataclasses.replace(cp, use_tc_tiling_on_sc=False)

    row_buf = lambda n: pltpu.VMEM((n, EMB_DIM), jnp.float32)

    @functools.partial(
        pl.kernel,
        compiler_params=cp,
        out_type=[
            jax.ShapeDtypeStruct((B,), jnp.float32),
            jax.ShapeDtypeStruct((B * NEG_K,), jnp.float32),
        ],
        mesh=mesh,
        scratch_types=[
            pltpu.VMEM((per_w,), jnp.int32),
            pltpu.VMEM((per_w,), jnp.int32),
            pltpu.VMEM((per_w * NEG_K,), jnp.int32),
            row_buf(CHUNK), row_buf(CHUNK), row_buf(NEG_ROWS),
            row_buf(CHUNK), row_buf(CHUNK), row_buf(NEG_ROWS),
            pltpu.VMEM((per_w,), jnp.float32),
            pltpu.VMEM((per_w * NEG_K,), jnp.float32),
            pltpu.SemaphoreType.DMA,
            pltpu.SemaphoreType.DMA,
        ],
    )
    def sc_kernel(pos_u_hbm, pos_v_hbm, neg_hbm, uw_hbm, vw_hbm,
                  pos_out, neg_out,
                  idxu_all, idxv_all, idxn_all,
                  ur_a, vr_a, nr_a, ur_b, vr_b, nr_b,
                  psco_all, nsco_all, sem_a, sem_b):
        wid = lax.axis_index("s") * NC + lax.axis_index("c")
        lane = lax.iota(jnp.int32, LANES)
        base_w = wid * per_w

        pltpu.sync_copy(pos_u_hbm.at[pl.ds(base_w, per_w)], idxu_all)
        pltpu.sync_copy(pos_v_hbm.at[pl.ds(base_w, per_w)], idxv_all)
        pltpu.sync_copy(neg_hbm.at[pl.ds(base_w * NEG_K, per_w * NEG_K)],
                        idxn_all)

        def issue(c, ur, vr, nr, sem):
            pltpu.async_copy(uw_hbm.at[idxu_all.at[pl.ds(c * CHUNK, CHUNK)]],
                             ur, sem)
            pltpu.async_copy(vw_hbm.at[idxv_all.at[pl.ds(c * CHUNK, CHUNK)]],
                             vr, sem)
            for j in range(NEG_ROWS // GATHER_W):
                pltpu.async_copy(
                    vw_hbm.at[idxn_all.at[pl.ds(c * NEG_ROWS + j * GATHER_W,
                                                GATHER_W)]],
                    nr.at[pl.ds(j * GATHER_W, GATHER_W)], sem)

        def drain(ur, vr, nr, sem):
            pltpu.make_async_copy(uw_hbm.at[pl.ds(0, CHUNK)], ur, sem).wait()
            pltpu.make_async_copy(uw_hbm.at[pl.ds(0, CHUNK)], vr, sem).wait()
            pltpu.make_async_copy(uw_hbm.at[pl.ds(0, NEG_ROWS)], nr, sem).wait()

        def compute(c, ur, vr, nr):
            @pl.loop(0, CHUNK // LANES)
            def _grp(g):
                urow = lane + g * LANES
                accp = jnp.zeros((LANES,), jnp.float32)
                colv = lane
                for _f in range(EMB_DIM):
                    gu = plsc.load_gather(ur, [urow, colv])
                    gv = plsc.load_gather(vr, [urow, colv])
                    accp = accp + gu * gv
                    colv = (colv + 1) & (EMB_DIM - 1)
                psco_all[pl.ds(c * CHUNK + g * LANES, LANES)] = accp

                sbase = (c * CHUNK + g * LANES) * NEG_K + lane * NEG_K

                @pl.loop(0, NEG_K, step=KQ)
                def _negs(k):
                    accs = [jnp.zeros((LANES,), jnp.float32)
                            for _ in range(KQ)]
                    nrow = [urow * NEG_K + (k + q) for q in range(KQ)]
                    colv = lane
                    for _f in range(EMB_DIM):
                        gu = plsc.load_gather(ur, [urow, colv])
                        for q in range(KQ):
                            gn = plsc.load_gather(nr, [nrow[q], colv])
                            accs[q] = accs[q] + gu * gn
                        colv = (colv + 1) & (EMB_DIM - 1)
                    for q in range(KQ):
                        plsc.store_scatter(nsco_all, [sbase + (k + q)],
                                           accs[q])

        issue(0, ur_a, vr_a, nr_a, sem_a)
        issue(1, ur_b, vr_b, nr_b, sem_b)

        @pl.loop(0, n_chunks // 2)
        def _pipe(i):
            c0 = i * 2
            drain(ur_a, vr_a, nr_a, sem_a)
            compute(c0, ur_a, vr_a, nr_a)

            @pl.when(c0 + 2 < n_chunks)
            def _():
                issue(c0 + 2, ur_a, vr_a, nr_a, sem_a)

            drain(ur_b, vr_b, nr_b, sem_b)
            compute(c0 + 1, ur_b, vr_b, nr_b)

            @pl.when(c0 + 3 < n_chunks)
            def _():
                issue(c0 + 3, ur_b, vr_b, nr_b, sem_b)

        pltpu.sync_copy(psco_all, pos_out.at[pl.ds(base_w, per_w)])
        pltpu.sync_copy(nsco_all,
                        neg_out.at[pl.ds(base_w * NEG_K, per_w * NEG_K)])

    return sc_kernel(pos_u, pos_v, neg_flat, u_weight, v_weight)


def _tc_loss(pos_s, neg_s, batch):
    def body(p_ref, n_ref, o_ref):
        s = jnp.clip(p_ref[...], -10.0, 10.0)
        t1 = jnp.sum(-jax.nn.log_sigmoid(s))
        ns = jnp.clip(n_ref[...], -10.0, 10.0)
        t2 = jnp.sum(-jax.nn.log_sigmoid(-ns))
        o_ref[...] = jnp.reshape((t1 + t2) / batch, (1, 1))

    return pl.pallas_call(
        body,
        out_shape=jax.ShapeDtypeStruct((1, 1), jnp.float32),
    )(pos_s, neg_s)


def kernel(pos_u, pos_v, neg_v, u_weight, v_weight):
    B = pos_u.shape[0]
    pos_u = pos_u.astype(jnp.int32)
    pos_v = pos_v.astype(jnp.int32)
    neg_flat = neg_v.astype(jnp.int32).reshape(-1)
    pos_s, neg_s = _sc_scores(pos_u, pos_v, neg_flat, u_weight, v_weight)
    loss = _tc_loss(pos_s, neg_s, float(B))
    return loss[0, 0]
